# baseline (speedup 1.0000x reference)
import jax
import jax.numpy as jnp
from jax.experimental import pallas as pl
from jax.experimental.pallas import tpu as pltpu

_HEADS = 4
_CHUNK = 2048


def _proj_kernel(x_ref, w_ref, b_ref, o_ref):
    i = pl.program_id(0)
    xc = x_ref[pl.ds(i * _CHUNK, _CHUNK), :].astype(jnp.bfloat16)
    wb = w_ref[...].astype(jnp.bfloat16)
    yt = jax.lax.dot_general(
        wb, xc, (((0,), (1,)), ((), ())), preferred_element_type=jnp.float32
    )
    o_ref[...] = yt + b_ref[...]


@jax.jit
def kernel(x, W, b):
    M, K = x.shape
    N = W.shape[1]
    b2 = b.reshape(N, 1)
    yt = pl.pallas_call(
        _proj_kernel,
        grid=(M // _CHUNK,),
        in_specs=[
            pl.BlockSpec(memory_space=pltpu.MemorySpace.VMEM),
            pl.BlockSpec(memory_space=pltpu.MemorySpace.VMEM),
            pl.BlockSpec(memory_space=pltpu.MemorySpace.VMEM),
        ],
        out_specs=pl.BlockSpec((N, _CHUNK), lambda i: (0, i)),
        out_shape=jax.ShapeDtypeStruct((N, M), jnp.float32),
        compiler_params=pltpu.CompilerParams(
            dimension_semantics=("arbitrary",),
        ),
    )(x, W, b2)
    return yt.reshape(_HEADS, N // _HEADS, M).transpose(2, 0, 1)
